# baseline (speedup 1.0000x reference)
import functools

import jax
import jax.numpy as jnp
from jax import lax
from jax.experimental import pallas as pl
from jax.experimental.pallas import tpu as pltpu
from jax.experimental.pallas import tpu_sc as plsc

_D = 256
_MAXPOS = 100
_NC, _NS = 2, 16
_NW = _NC * _NS
_CH = 32
_NBUF = 4
_LANES = 16


def _sc_body(x_hbm, pos_hbm, tab_hbm, out_hbm, post, idx_all,
             xv, rv, in_sems, g_sems, o_sems):
    wid = lax.axis_index("s") * _NC + lax.axis_index("c")
    b_per_w = x_hbm.shape[0] // _NW
    n_chunks = b_per_w // _CH
    base_w = wid * b_per_w

    pltpu.sync_copy(pos_hbm.at[pl.ds(2 * base_w, 2 * b_per_w)], post)
    lane = lax.iota(jnp.int32, _LANES)
    iv0 = 2 * (lane % 8)
    iv1 = iv0 + 1
    lo = lane < 8

    def _lgather(vec, iv):
        return vec.at[iv].get(mode="promise_in_bounds", unique_indices=True)

    def mk_idx(c, carry):
        for u in range(_CH // _LANES):
            s = (c * _CH + u * _LANES) * 2
            a = post[pl.ds(s, _LANES)]
            b = post[pl.ds(s + _LANES, _LANES)]
            p0 = jnp.where(lo, _lgather(a, iv0), _lgather(b, iv0))
            p1 = jnp.where(lo, _lgather(a, iv1), _lgather(b, iv1))
            idx_all[c, pl.ds(u * _LANES, _LANES)] = p0 * _MAXPOS + p1
        return carry

    lax.fori_loop(0, n_chunks, mk_idx, 0)

    def issue_in(c, b):
        base = base_w + c * _CH
        pltpu.async_copy(x_hbm.at[pl.ds(base, _CH)], xv[b], in_sems[b])
        pltpu.async_copy(tab_hbm.at[idx_all.at[c]], rv[b], g_sems[b])

    for b in range(_NBUF - 1):
        issue_in(b, b)

    def group(g, carry):
        for b in range(_NBUF):
            c = g * _NBUF + b
            s3 = (b + _NBUF - 1) % _NBUF

            @pl.when(c + _NBUF - 1 < n_chunks)
            def _refill():
                @pl.when(c >= 1)
                def _drain():
                    pltpu.make_async_copy(
                        rv[s3], out_hbm.at[pl.ds(base_w, _CH)],
                        o_sems[s3]).wait()
                issue_in(c + _NBUF - 1, s3)

            pltpu.make_async_copy(
                x_hbm.at[pl.ds(base_w, _CH)], xv[b], in_sems[b]).wait()
            pltpu.make_async_copy(
                tab_hbm.at[idx_all.at[c]], rv[b], g_sems[b]).wait()

            def add_row(r, carry2):
                for u in range(_D // _LANES):
                    d = pl.ds(u * _LANES, _LANES)
                    plsc.addupdate(rv[b].at[r, d], xv[b][r, d])
                return carry2

            lax.fori_loop(0, _CH, add_row, 0)
            pltpu.async_copy(
                rv[b], out_hbm.at[pl.ds(base_w + c * _CH, _CH)], o_sems[b])
        return carry

    lax.fori_loop(0, n_chunks // _NBUF, group, 0)

    for b in range(_NBUF):
        pltpu.make_async_copy(
            rv[b], out_hbm.at[pl.ds(base_w, _CH)], o_sems[b]).wait()


@jax.jit
def _run(x2, pos2, tab):
    B = x2.shape[0]
    b_per_w = B // _NW
    n_chunks = b_per_w // _CH
    mesh = plsc.VectorSubcoreMesh(core_axis_name="c", subcore_axis_name="s")
    k = pl.kernel(
        _sc_body,
        out_type=jax.ShapeDtypeStruct((B, _D), jnp.float32),
        mesh=mesh,
        scratch_types=[
            pltpu.VMEM((2 * b_per_w,), jnp.int32),
            pltpu.VMEM((n_chunks, _CH), jnp.int32),
            [pltpu.VMEM((_CH, _D), jnp.float32) for _ in range(_NBUF)],
            [pltpu.VMEM((_CH, _D), jnp.float32) for _ in range(_NBUF)],
            [pltpu.SemaphoreType.DMA for _ in range(_NBUF)],
            [pltpu.SemaphoreType.DMA for _ in range(_NBUF)],
            [pltpu.SemaphoreType.DMA for _ in range(_NBUF)],
        ],
    )
    return k(x2, pos2, tab)


def kernel(x, pos, pos_embeddings):
    b, t, d = x.shape
    B = b * t
    x2 = x.reshape(B, d)
    pos2 = pos.astype(jnp.int32).reshape(2 * B)
    tab = pos_embeddings.reshape(-1, d)
    return _run(x2, pos2, tab).reshape(b, t, d)


# speedup vs baseline: 1.0020x; 1.0020x over previous
import functools

import jax
import jax.numpy as jnp
from jax import lax
from jax.experimental import pallas as pl
from jax.experimental.pallas import tpu as pltpu
from jax.experimental.pallas import tpu_sc as plsc

_D = 256
_MAXPOS = 100
_NC, _NS = 2, 16
_NW = _NC * _NS
_CH = 32
_NBUF = 4
_LANES = 16


def _sc_body(x_hbm, pos_hbm, tab_hbm, out_hbm, post, idx_all,
             xv, rv, in_sems, g_sems, o_sems):
    wid = lax.axis_index("s") * _NC + lax.axis_index("c")
    b_per_w = x_hbm.shape[0] // _NW
    n_chunks = b_per_w // _CH
    base_w = wid * b_per_w

    pltpu.sync_copy(pos_hbm.at[pl.ds(2 * base_w, 2 * b_per_w)], post)
    lane = lax.iota(jnp.int32, _LANES)
    iv0 = 2 * (lane % 8)
    iv1 = iv0 + 1
    lo = lane < 8

    def _lgather(vec, iv):
        return vec.at[iv].get(mode="promise_in_bounds", unique_indices=True)

    def mk_idx(c, carry):
        for u in range(_CH // _LANES):
            s = (c * _CH + u * _LANES) * 2
            a = post[pl.ds(s, _LANES)]
            b = post[pl.ds(s + _LANES, _LANES)]
            p0 = jnp.where(lo, _lgather(a, iv0), _lgather(b, iv0))
            p1 = jnp.where(lo, _lgather(a, iv1), _lgather(b, iv1))
            idx_all[c, pl.ds(u * _LANES, _LANES)] = p0 * _MAXPOS + p1
        return carry

    lax.fori_loop(0, n_chunks, mk_idx, 0)

    def issue_in(c, b):
        base = base_w + c * _CH
        pltpu.async_copy(x_hbm.at[pl.ds(base, _CH)], xv[b], in_sems[b])
        pltpu.async_copy(tab_hbm.at[idx_all.at[c]], rv[b], g_sems[b])

    for b in range(_NBUF - 1):
        issue_in(b, b)

    def group(g, carry):
        for b in range(_NBUF):
            c = g * _NBUF + b
            s3 = (b + _NBUF - 1) % _NBUF

            @pl.when(c + _NBUF - 1 < n_chunks)
            def _refill():
                @pl.when(c >= 1)
                def _drain():
                    pltpu.make_async_copy(
                        rv[s3], out_hbm.at[pl.ds(base_w, _CH)],
                        o_sems[s3]).wait()
                issue_in(c + _NBUF - 1, s3)

            pltpu.make_async_copy(
                x_hbm.at[pl.ds(base_w, _CH)], xv[b], in_sems[b]).wait()
            pltpu.make_async_copy(
                tab_hbm.at[idx_all.at[c]], rv[b], g_sems[b]).wait()

            def add_row(r, carry2):
                for u in range(_D // _LANES):
                    d = pl.ds(u * _LANES, _LANES)
                    rv[b][r, d] = rv[b][r, d] + xv[b][r, d]
                return carry2

            lax.fori_loop(0, _CH, add_row, 0)
            pltpu.async_copy(
                rv[b], out_hbm.at[pl.ds(base_w + c * _CH, _CH)], o_sems[b])
        return carry

    lax.fori_loop(0, n_chunks // _NBUF, group, 0)

    for b in range(_NBUF):
        pltpu.make_async_copy(
            rv[b], out_hbm.at[pl.ds(base_w, _CH)], o_sems[b]).wait()


@jax.jit
def _run(x2, pos2, tab):
    B = x2.shape[0]
    b_per_w = B // _NW
    n_chunks = b_per_w // _CH
    mesh = plsc.VectorSubcoreMesh(core_axis_name="c", subcore_axis_name="s")
    k = pl.kernel(
        _sc_body,
        out_type=jax.ShapeDtypeStruct((B, _D), jnp.float32),
        mesh=mesh,
        scratch_types=[
            pltpu.VMEM((2 * b_per_w,), jnp.int32),
            pltpu.VMEM((n_chunks, _CH), jnp.int32),
            [pltpu.VMEM((_CH, _D), jnp.float32) for _ in range(_NBUF)],
            [pltpu.VMEM((_CH, _D), jnp.float32) for _ in range(_NBUF)],
            [pltpu.SemaphoreType.DMA for _ in range(_NBUF)],
            [pltpu.SemaphoreType.DMA for _ in range(_NBUF)],
            [pltpu.SemaphoreType.DMA for _ in range(_NBUF)],
        ],
    )
    return k(x2, pos2, tab)


def kernel(x, pos, pos_embeddings):
    b, t, d = x.shape
    B = b * t
    x2 = x.reshape(B, d)
    pos2 = pos.astype(jnp.int32).reshape(2 * B)
    tab = pos_embeddings.reshape(-1, d)
    return _run(x2, pos2, tab).reshape(b, t, d)
